# baseline (speedup 1.0000x reference)
import functools

import jax
import jax.numpy as jnp
from jax import lax
from jax.experimental import pallas as pl
from jax.experimental.pallas import tpu as pltpu
from jax.experimental.pallas import tpu_sc as plsc

VOCAB = 100000
D = 128
B = 4096
S = 50
TI_LEN = 8
TI_START = 11

NC, NS = 2, 16
NW = NC * NS
N_ROWS = S * B
COLS_PER_W = B // NW

GATHER_S = [s for s in range(S) if not TI_START <= s < TI_START + TI_LEN]
NBUF = 3
FILL_ROWS = 16
N_FILL_W = COLS_PER_W // FILL_ROWS


@functools.cache
def _build_sc_embed():
    mesh = plsc.VectorSubcoreMesh(
        core_axis_name="c", subcore_axis_name="s", num_cores=NC, num_subcores=NS
    )

    @functools.partial(
        pl.kernel,
        mesh=mesh,
        out_type=jax.ShapeDtypeStruct((N_ROWS, D), jnp.float32),
        scratch_types=[
            pltpu.VMEM((S, COLS_PER_W), jnp.int32),
            pltpu.VMEM((TI_LEN, D), jnp.float32),
            [pltpu.VMEM((FILL_ROWS, D), jnp.float32) for _ in range(TI_LEN)],
            [pltpu.VMEM((COLS_PER_W, D), jnp.float32) for _ in range(NBUF)],
            [pltpu.SemaphoreType.DMA for _ in range(NBUF)],
            [pltpu.SemaphoreType.DMA for _ in range(NBUF)],
            pltpu.SemaphoreType.DMA,
        ],
    )
    def _sc_embed(
        ids_hbm, table_hbm, ti_hbm, out_hbm,
        idx_v, ti_v, fills, bufs, gsems, wsems, tsem,
    ):
        wid = lax.axis_index("s") * NC + lax.axis_index("c")
        col0 = wid * COLS_PER_W

        pltpu.sync_copy(ti_hbm, ti_v)
        for t in range(TI_LEN):
            for c8 in range(D // 16):
                v = ti_v[t, pl.ds(c8 * 16, 16)]
                for r in range(FILL_ROWS):
                    fills[t][r, pl.ds(c8 * 16, 16)] = v
        tds = [
            pltpu.async_copy(
                fills[t],
                out_hbm.at[
                    pl.ds((TI_START + t) * B + col0 + j * FILL_ROWS, FILL_ROWS)
                ],
                tsem,
            )
            for t in range(TI_LEN)
            for j in range(N_FILL_W)
        ]

        pltpu.sync_copy(ids_hbm.at[:, pl.ds(col0, COLS_PER_W)], idx_v)

        def gather(k):
            return pltpu.async_copy(
                table_hbm.at[idx_v.at[GATHER_S[k]]], bufs[k % NBUF], gsems[k % NBUF]
            )

        def write(k):
            return pltpu.async_copy(
                bufs[k % NBUF],
                out_hbm.at[pl.ds(GATHER_S[k] * B + col0, COLS_PER_W)],
                wsems[k % NBUF],
            )

        n = len(GATHER_S)
        gds = {k: gather(k) for k in range(NBUF)}
        pending_w = {}
        for k in range(n):
            gds.pop(k).wait()
            pending_w[k] = write(k)
            if k + NBUF < n:
                pending_w.pop(k).wait()
                gds[k + NBUF] = gather(k + NBUF)
        for k in sorted(pending_w):
            pending_w[k].wait()
        for td in tds:
            td.wait()

    return _sc_embed


def kernel(input_ids, table, ti_vec, out_dtype):
    del out_dtype
    ids_t = input_ids.astype(jnp.int32).T
    out_flat = _build_sc_embed()(ids_t, table, ti_vec)
    return jnp.transpose(out_flat.reshape(S, B, D), (1, 0, 2))


# speedup vs baseline: 1.0388x; 1.0388x over previous
import functools

import jax
import jax.numpy as jnp
from jax import lax
from jax.experimental import pallas as pl
from jax.experimental.pallas import tpu as pltpu
from jax.experimental.pallas import tpu_sc as plsc

VOCAB = 100000
D = 128
B = 4096
S = 50
TI_LEN = 8
TI_START = 11

NC, NS = 2, 16
NW = NC * NS
N_ROWS = S * B

TI_LO = TI_START * B
TI_HI = (TI_START + TI_LEN) * B

A_PER_W = TI_LO // NW
B_PER_W = (N_ROWS - TI_HI) // NW
G_PER_W = A_PER_W + B_PER_W
TI_PER_W = (TI_HI - TI_LO) // NW

CHUNK = 448
N_CHUNKS = G_PER_W // CHUNK
FILL_ROWS = 32
N_FILL = TI_PER_W // FILL_ROWS


@functools.cache
def _build_sc_embed():
    mesh = plsc.VectorSubcoreMesh(
        core_axis_name="c", subcore_axis_name="s", num_cores=NC, num_subcores=NS
    )

    @functools.partial(
        pl.kernel,
        mesh=mesh,
        out_type=jax.ShapeDtypeStruct((N_ROWS, D), jnp.float32),
        scratch_types=[
            pltpu.VMEM((G_PER_W,), jnp.int32),
            pltpu.VMEM((FILL_ROWS, D), jnp.float32),
            pltpu.VMEM((1, D), jnp.float32),
            pltpu.VMEM((CHUNK, D), jnp.float32),
            pltpu.VMEM((CHUNK, D), jnp.float32),
            pltpu.SemaphoreType.DMA,
            pltpu.SemaphoreType.DMA,
            pltpu.SemaphoreType.DMA,
            pltpu.SemaphoreType.DMA,
            pltpu.SemaphoreType.DMA,
        ],
    )
    def _sc_embed(
        ids_hbm, table_hbm, ti_hbm, out_hbm,
        idx_v, fill_v, tirow_v, buf0, buf1, g0, g1, w0, w1, tsem,
    ):
        wid = lax.axis_index("s") * NC + lax.axis_index("c")
        bufs, gsems, wsems = (buf0, buf1), (g0, g1), (w0, w1)
        a0 = wid * A_PER_W
        b0 = TI_HI + wid * B_PER_W

        s_off = wid // (NW // TI_LEN)
        pltpu.sync_copy(ti_hbm.at[s_off], tirow_v.at[0])
        for c8 in range(D // 16):
            v = tirow_v[0, pl.ds(c8 * 16, 16)]
            for r in range(FILL_ROWS):
                fill_v[r, pl.ds(c8 * 16, 16)] = v
        ti0 = TI_LO + wid * TI_PER_W
        tds = [
            pltpu.async_copy(
                fill_v, out_hbm.at[pl.ds(ti0 + j * FILL_ROWS, FILL_ROWS)], tsem
            )
            for j in range(N_FILL)
        ]

        pltpu.sync_copy(ids_hbm.at[pl.ds(a0, A_PER_W)], idx_v.at[pl.ds(0, A_PER_W)])
        pltpu.sync_copy(
            ids_hbm.at[pl.ds(b0, B_PER_W)], idx_v.at[pl.ds(A_PER_W, B_PER_W)]
        )

        def gather(k):
            return pltpu.async_copy(
                table_hbm.at[idx_v.at[pl.ds(k * CHUNK, CHUNK)]],
                bufs[k % 2],
                gsems[k % 2],
            )

        def write(k):
            v0 = k * CHUNK
            wds = []
            if v0 < A_PER_W < v0 + CHUNK:
                la = A_PER_W - v0
                wds.append(
                    pltpu.async_copy(
                        bufs[k % 2].at[pl.ds(0, la)],
                        out_hbm.at[pl.ds(a0 + v0, la)],
                        wsems[k % 2],
                    )
                )
                wds.append(
                    pltpu.async_copy(
                        bufs[k % 2].at[pl.ds(la, CHUNK - la)],
                        out_hbm.at[pl.ds(b0, CHUNK - la)],
                        wsems[k % 2],
                    )
                )
            else:
                off = a0 + v0 if v0 + CHUNK <= A_PER_W else b0 + (v0 - A_PER_W)
                wds.append(
                    pltpu.async_copy(
                        bufs[k % 2],
                        out_hbm.at[pl.ds(off, CHUNK)],
                        wsems[k % 2],
                    )
                )
            return wds

        gds = {0: gather(0), 1: gather(1)}
        pending_w = {}
        for k in range(N_CHUNKS):
            gds.pop(k).wait()
            pending_w[k] = write(k)
            if k + 2 < N_CHUNKS:
                for wd in pending_w.pop(k):
                    wd.wait()
                gds[k + 2] = gather(k + 2)
        for k, wds in sorted(pending_w.items()):
            for wd in wds:
                wd.wait()
        for td in tds:
            td.wait()

    return _sc_embed


def kernel(input_ids, table, ti_vec, out_dtype):
    del out_dtype
    ids_t = input_ids.astype(jnp.int32).T.reshape(N_ROWS)
    out_flat = _build_sc_embed()(ids_t, table, ti_vec)
    return jnp.transpose(out_flat.reshape(S, B, D), (1, 0, 2))
